# Initial kernel scaffold; baseline (speedup 1.0000x reference)
import jax
import jax.numpy as jnp
from jax.experimental import pallas as pl


def kernel(query, key, value, Wq, bq, Wv, bv, Wout, bout, hash_w):
    raise NotImplementedError("write your pallas kernel here")


# speedup vs baseline: 1.3227x; 1.3227x over previous
import functools
import jax
import jax.numpy as jnp
from jax import lax
from jax.experimental import pallas as pl
from jax.experimental.pallas import tpu as pltpu
from jax.experimental.pallas import tpu_sc as plsc

C = 128
W = 128


def _proj_hash_kernel(x_ref, val_ref, wqt_ref, bq_ref, wvt_ref, bv_ref, hw_ref,
                      q_ref, v_ref, codes_ref, *, R, H, NHh):
    xb = x_ref[...]
    qb = jnp.dot(xb, wqt_ref[...], preferred_element_type=jnp.float32) + bq_ref[...]
    vb = jnp.dot(val_ref[...], wvt_ref[...], preferred_element_type=jnp.float32) + bv_ref[...]
    q_ref[...] = qb
    v_ref[...] = vb
    rot = jnp.dot(qb, hw_ref[...], preferred_element_type=jnp.float32)
    rows = rot.shape[0]
    rot4 = rot.reshape(rows, R, H, NHh)
    iota = lax.broadcasted_iota(jnp.int32, rot4.shape, 3)
    amax = jnp.max(rot4, axis=-1, keepdims=True)
    ia = jnp.min(jnp.where(rot4 >= amax, iota, NHh), axis=-1)
    amin = jnp.min(rot4, axis=-1, keepdims=True)
    ib = jnp.min(jnp.where(rot4 <= amin, iota, NHh), axis=-1)
    code = jnp.where(amax[..., 0] >= -amin[..., 0], ia, NHh + ib)
    codes_ref[...] = code.reshape(rows, R * H)


def _attn_kernel(q_ref, qp_ref, v_ref, vp_ref, p_ref, pp_ref, h_ref, hp_ref,
                 olse_ref, *, scaling, Dh):
    qc = q_ref[0]
    qp = qp_ref[0]

    def norm(x):
        return x / (jnp.sqrt(jnp.sum(x * x, axis=-1, keepdims=True)) + 1e-6)

    k2 = jnp.concatenate([norm(qc), norm(qp)], axis=0)
    v2 = jnp.concatenate([v_ref[0], vp_ref[0]], axis=0)
    pc = p_ref[0, 0]
    p2 = jnp.concatenate([pc, pp_ref[0, 0]], axis=0)
    hc = h_ref[0, 0]
    h2 = jnp.concatenate([hc, hp_ref[0, 0]], axis=0)
    scores = jnp.dot(qc, k2.T, preferred_element_type=jnp.float32) * scaling
    self_m = pc[:, None] == p2[None, :]
    diff_m = hc[:, None] != h2[None, :]
    scores = jnp.where(self_m, -1e8, scores)
    scores = jnp.where(diff_m, -1e16, scores)
    m = jnp.max(scores, axis=-1, keepdims=True)
    p = jnp.exp(scores - m)
    denom = jnp.sum(p, axis=-1, keepdims=True)
    o = jnp.dot(p, v2, preferred_element_type=jnp.float32) / denom
    lse = m + jnp.log(denom)
    pad = jnp.zeros((o.shape[0], W - Dh - 1), jnp.float32)
    olse_ref[0] = jnp.concatenate([o, lse, pad], axis=1)


def _sc_unsort_gather(table, idx, per_w, ch, nch):
    mesh = plsc.VectorSubcoreMesh(core_axis_name="c", subcore_axis_name="s")

    @functools.partial(
        pl.kernel, mesh=mesh,
        out_type=jax.ShapeDtypeStruct(table.shape, jnp.float32),
        scratch_types=[
            pltpu.VMEM((ch,), jnp.int32),
            pltpu.VMEM((ch, table.shape[1]), jnp.float32),
            pltpu.SemaphoreType.DMA,
        ],
    )
    def k(table_hbm, idx_hbm, out_hbm, idx_v, rows_v, sem):
        cid = lax.axis_index("c")
        sid = lax.axis_index("s")
        wid = sid * 2 + cid
        base = wid * per_w
        for ci in range(nch):
            off = base + ci * ch
            pltpu.sync_copy(idx_hbm.at[pl.ds(off, ch)], idx_v)
            pltpu.async_copy(table_hbm.at[idx_v], rows_v, sem).wait()
            pltpu.sync_copy(rows_v, out_hbm.at[pl.ds(off, ch)])

    return k(table, idx)


def _combine_kernel(g_ref, wot_ref, bo_ref, out_ref, *, H, Dh):
    g = g_ref[...]
    R_, rows, _ = g.shape
    g4 = g.reshape(R_, rows, H, W)
    o = g4[..., :Dh].reshape(R_, rows, H * Dh)
    l = g4[..., Dh]
    m = jnp.max(l, axis=0, keepdims=True)
    w = jnp.exp(l - m)
    w = w / jnp.sum(w, axis=0, keepdims=True)
    wexp = jnp.broadcast_to(w[..., None], (R_, rows, H, Dh)).reshape(R_, rows, H * Dh)
    comb = jnp.sum(wexp * o, axis=0)
    out_ref[...] = jnp.dot(comb, wot_ref[...], preferred_element_type=jnp.float32) + bo_ref[...]


def kernel(query, key, value, Wq, bq, Wv, bv, Wout, bout, hash_w):
    T, B, E = query.shape
    R, H, Dh, NHh = hash_w.shape
    nC = T // C
    rows = T * B
    scaling = Dh ** -0.5

    eye = jnp.eye(H, dtype=jnp.float32)
    HW = jnp.einsum('rhdm,hg->hdrgm', hash_w, eye).reshape(E, R * H * NHh)

    x2d = query.reshape(rows, E)
    val2d = value.reshape(rows, E)
    BLK = 256
    nblk = rows // BLK

    q2d, v2d, codes2d = pl.pallas_call(
        functools.partial(_proj_hash_kernel, R=R, H=H, NHh=NHh),
        grid=(nblk,),
        in_specs=[
            pl.BlockSpec((BLK, E), lambda i: (i, 0)),
            pl.BlockSpec((BLK, E), lambda i: (i, 0)),
            pl.BlockSpec((E, E), lambda i: (0, 0)),
            pl.BlockSpec((1, E), lambda i: (0, 0)),
            pl.BlockSpec((E, E), lambda i: (0, 0)),
            pl.BlockSpec((1, E), lambda i: (0, 0)),
            pl.BlockSpec((E, R * H * NHh), lambda i: (0, 0)),
        ],
        out_specs=[
            pl.BlockSpec((BLK, E), lambda i: (i, 0)),
            pl.BlockSpec((BLK, E), lambda i: (i, 0)),
            pl.BlockSpec((BLK, R * H), lambda i: (i, 0)),
        ],
        out_shape=[
            jax.ShapeDtypeStruct((rows, E), jnp.float32),
            jax.ShapeDtypeStruct((rows, E), jnp.float32),
            jax.ShapeDtypeStruct((rows, R * H), jnp.int32),
        ],
    )(x2d, val2d, Wq.T, bq.reshape(1, E), Wv.T, bv.reshape(1, E), HW)

    codes = codes2d.reshape(T, B, R, H).transpose(2, 1, 3, 0)

    pos = jnp.arange(T, dtype=jnp.int32)
    keys = codes * T + pos[None, None, None, :]
    skey = jnp.sort(keys, axis=-1)
    sidx = (skey % T).astype(jnp.int32)
    sh = skey // T
    undo = jnp.argsort(sidx, axis=-1).astype(jnp.int32)

    q_bhtd = q2d.reshape(T, B, H, Dh).transpose(1, 2, 0, 3)
    v_bhtd = v2d.reshape(T, B, H, Dh).transpose(1, 2, 0, 3)
    sq = jnp.take_along_axis(q_bhtd[None], sidx[..., None], axis=3)
    sv = jnp.take_along_axis(v_bhtd[None], sidx[..., None], axis=3)
    sp = sidx

    G = R * B * H * nC
    sq4 = sq.reshape(G, C, Dh)
    sv4 = sv.reshape(G, C, Dh)
    sp3 = sp.reshape(G, 1, C)
    sh3 = sh.reshape(G, 1, C)

    def cur(g):
        return (g, 0, 0)

    def prev(g):
        return ((g // nC) * nC + (g % nC + nC - 1) % nC, 0, 0)

    olse = pl.pallas_call(
        functools.partial(_attn_kernel, scaling=scaling, Dh=Dh),
        grid=(G,),
        in_specs=[
            pl.BlockSpec((1, C, Dh), cur),
            pl.BlockSpec((1, C, Dh), prev),
            pl.BlockSpec((1, C, Dh), cur),
            pl.BlockSpec((1, C, Dh), prev),
            pl.BlockSpec((1, 1, C), cur),
            pl.BlockSpec((1, 1, C), prev),
            pl.BlockSpec((1, 1, C), cur),
            pl.BlockSpec((1, 1, C), prev),
        ],
        out_specs=pl.BlockSpec((1, C, W), cur),
        out_shape=jax.ShapeDtypeStruct((G, C, W), jnp.float32),
    )(sq4, sq4, sv4, sv4, sp3, sp3, sh3, sh3)

    N = R * B * H * T
    table = olse.reshape(N, W)
    undo_m = jnp.moveaxis(undo, 3, 1)
    bidx = jnp.arange(B, dtype=jnp.int32)
    hidx = jnp.arange(H, dtype=jnp.int32)
    ridx = jnp.arange(R, dtype=jnp.int32)
    base = ((ridx[:, None, None] * B + bidx[None, :, None]) * H
            + hidx[None, None, :]) * T
    gidx = (undo_m + base[:, None, :, :]).reshape(N)

    NW = 32
    per_w = N // NW
    ch = 512
    nch = per_w // ch
    gathered = _sc_unsort_gather(table, gidx, per_w, ch, nch)

    g3 = gathered.reshape(R, rows, H * W)

    out2d = pl.pallas_call(
        functools.partial(_combine_kernel, H=H, Dh=Dh),
        grid=(nblk,),
        in_specs=[
            pl.BlockSpec((R, BLK, H * W), lambda i: (0, i, 0)),
            pl.BlockSpec((E, E), lambda i: (0, 0)),
            pl.BlockSpec((1, E), lambda i: (0, 0)),
        ],
        out_specs=pl.BlockSpec((BLK, E), lambda i: (i, 0)),
        out_shape=jax.ShapeDtypeStruct((rows, E), jnp.float32),
    )(g3, Wout.T, bout.reshape(1, E))

    return out2d.reshape(T, B, E)


# speedup vs baseline: 5.5568x; 4.2010x over previous
import functools
import jax
import jax.numpy as jnp
from jax import lax
from jax.experimental import pallas as pl
from jax.experimental.pallas import tpu as pltpu
from jax.experimental.pallas import tpu_sc as plsc

C = 128
W = 128


def _proj_hash_kernel(x_ref, val_ref, wqt_ref, bq_ref, wvt_ref, bv_ref, hw_ref,
                      qv_ref, codes_ref, *, R, H, NHh, Dh):
    xb = x_ref[...]
    qb = jnp.dot(xb, wqt_ref[...], preferred_element_type=jnp.float32) + bq_ref[...]
    vb = jnp.dot(val_ref[...], wvt_ref[...], preferred_element_type=jnp.float32) + bv_ref[...]
    rows = rot_rows = xb.shape[0]
    qv = jnp.concatenate([qb.reshape(rows, H, Dh), vb.reshape(rows, H, Dh)], axis=-1)
    qv_ref[...] = qv.reshape(rows, H * 2 * Dh)
    rot = jnp.dot(qb, hw_ref[...], preferred_element_type=jnp.float32)
    rot4 = rot.reshape(rows, R, H, NHh)
    iota = lax.broadcasted_iota(jnp.int32, rot4.shape, 3)
    amax = jnp.max(rot4, axis=-1, keepdims=True)
    ia = jnp.min(jnp.where(rot4 >= amax, iota, NHh), axis=-1)
    amin = jnp.min(rot4, axis=-1, keepdims=True)
    ib = jnp.min(jnp.where(rot4 <= amin, iota, NHh), axis=-1)
    code = jnp.where(amax[..., 0] >= -amin[..., 0], ia, NHh + ib)
    codes_ref[...] = code.reshape(rows, R * H)


def _attn_kernel(qv_ref, qvp_ref, p_ref, pp_ref, h_ref, hp_ref,
                 olse_ref, *, scaling, Dh):
    qc = qv_ref[0][:, :Dh]
    qp = qvp_ref[0][:, :Dh]

    def norm(x):
        return x / (jnp.sqrt(jnp.sum(x * x, axis=-1, keepdims=True)) + 1e-6)

    k2 = jnp.concatenate([norm(qc), norm(qp)], axis=0)
    v2 = jnp.concatenate([qv_ref[0][:, Dh:], qvp_ref[0][:, Dh:]], axis=0)
    pc = p_ref[0, 0]
    p2 = jnp.concatenate([pc, pp_ref[0, 0]], axis=0)
    hc = h_ref[0, 0]
    h2 = jnp.concatenate([hc, hp_ref[0, 0]], axis=0)
    scores = jnp.dot(qc, k2.T, preferred_element_type=jnp.float32) * scaling
    self_m = pc[:, None] == p2[None, :]
    diff_m = hc[:, None] != h2[None, :]
    scores = jnp.where(self_m, -1e8, scores)
    scores = jnp.where(diff_m, -1e16, scores)
    m = jnp.max(scores, axis=-1, keepdims=True)
    p = jnp.exp(scores - m)
    denom = jnp.sum(p, axis=-1, keepdims=True)
    o = jnp.dot(p, v2, preferred_element_type=jnp.float32) / denom
    lse = m + jnp.log(denom)
    pad = jnp.zeros((o.shape[0], W - Dh - 1), jnp.float32)
    olse_ref[0] = jnp.concatenate([o, lse, pad], axis=1)


def _sc_unsort_gather(table, idx, per_w, ch, nch):
    mesh = plsc.VectorSubcoreMesh(core_axis_name="c", subcore_axis_name="s")

    @functools.partial(
        pl.kernel, mesh=mesh,
        out_type=jax.ShapeDtypeStruct((idx.shape[0], table.shape[1]), jnp.float32),
        scratch_types=[
            pltpu.VMEM((ch,), jnp.int32),
            pltpu.VMEM((ch, table.shape[1]), jnp.float32),
            pltpu.SemaphoreType.DMA,
        ],
    )
    def k(table_hbm, idx_hbm, out_hbm, idx_v, rows_v, sem):
        cid = lax.axis_index("c")
        sid = lax.axis_index("s")
        wid = sid * 2 + cid
        base = wid * per_w
        for ci in range(nch):
            off = base + ci * ch
            pltpu.sync_copy(idx_hbm.at[pl.ds(off, ch)], idx_v)
            pltpu.async_copy(table_hbm.at[idx_v], rows_v, sem).wait()
            pltpu.sync_copy(rows_v, out_hbm.at[pl.ds(off, ch)])

    return k(table, idx)


def _combine_kernel(g_ref, wot_ref, bo_ref, out_ref, *, H, Dh):
    g = g_ref[...]
    R_, rows, _ = g.shape
    g4 = g.reshape(R_, rows, H, W)
    o = g4[..., :Dh].reshape(R_, rows, H * Dh)
    l = g4[..., Dh]
    m = jnp.max(l, axis=0, keepdims=True)
    w = jnp.exp(l - m)
    w = w / jnp.sum(w, axis=0, keepdims=True)
    wexp = jnp.broadcast_to(w[..., None], (R_, rows, H, Dh)).reshape(R_, rows, H * Dh)
    comb = jnp.sum(wexp * o, axis=0)
    out_ref[...] = jnp.dot(comb, wot_ref[...], preferred_element_type=jnp.float32) + bo_ref[...]


def kernel(query, key, value, Wq, bq, Wv, bv, Wout, bout, hash_w):
    T, B, E = query.shape
    R, H, Dh, NHh = hash_w.shape
    nC = T // C
    rows = T * B
    scaling = Dh ** -0.5

    eye = jnp.eye(H, dtype=jnp.float32)
    HW = jnp.einsum('rhdm,hg->hdrgm', hash_w, eye).reshape(E, R * H * NHh)

    x2d = query.reshape(rows, E)
    val2d = value.reshape(rows, E)
    BLK = 256
    nblk = rows // BLK

    qv2d, codes2d = pl.pallas_call(
        functools.partial(_proj_hash_kernel, R=R, H=H, NHh=NHh, Dh=Dh),
        grid=(nblk,),
        in_specs=[
            pl.BlockSpec((BLK, E), lambda i: (i, 0)),
            pl.BlockSpec((BLK, E), lambda i: (i, 0)),
            pl.BlockSpec((E, E), lambda i: (0, 0)),
            pl.BlockSpec((1, E), lambda i: (0, 0)),
            pl.BlockSpec((E, E), lambda i: (0, 0)),
            pl.BlockSpec((1, E), lambda i: (0, 0)),
            pl.BlockSpec((E, R * H * NHh), lambda i: (0, 0)),
        ],
        out_specs=[
            pl.BlockSpec((BLK, H * 2 * Dh), lambda i: (i, 0)),
            pl.BlockSpec((BLK, R * H), lambda i: (i, 0)),
        ],
        out_shape=[
            jax.ShapeDtypeStruct((rows, H * 2 * Dh), jnp.float32),
            jax.ShapeDtypeStruct((rows, R * H), jnp.int32),
        ],
    )(x2d, val2d, Wq.T, bq.reshape(1, E), Wv.T, bv.reshape(1, E), HW)

    codes = codes2d.reshape(T, B, R, H).transpose(2, 1, 3, 0)

    pos = jnp.arange(T, dtype=jnp.int32)
    keys = codes * T + pos[None, None, None, :]
    skey = jnp.sort(keys, axis=-1)
    sidx = (skey % T).astype(jnp.int32)
    sh = skey // T
    undo = jnp.argsort(sidx, axis=-1).astype(jnp.int32)

    N2 = R * B * H * T
    bidx2 = jnp.arange(B, dtype=jnp.int32)
    hidx2 = jnp.arange(H, dtype=jnp.int32)
    sgidx = (sidx * (B * H)
             + (bidx2[None, :, None, None] * H + hidx2[None, None, :, None])
             ).reshape(N2)
    per_w2 = N2 // 32
    ch2 = 512
    sqv_rows = _sc_unsort_gather(qv2d.reshape(T * B * H, 2 * Dh), sgidx,
                                 per_w2, ch2, per_w2 // ch2)
    sp = sidx

    G = R * B * H * nC
    sqv4 = sqv_rows.reshape(G, C, 2 * Dh)
    sp3 = sp.reshape(G, 1, C)
    sh3 = sh.reshape(G, 1, C)

    def cur(g):
        return (g, 0, 0)

    def prev(g):
        return ((g // nC) * nC + (g % nC + nC - 1) % nC, 0, 0)

    olse = pl.pallas_call(
        functools.partial(_attn_kernel, scaling=scaling, Dh=Dh),
        grid=(G,),
        in_specs=[
            pl.BlockSpec((1, C, 2 * Dh), cur),
            pl.BlockSpec((1, C, 2 * Dh), prev),
            pl.BlockSpec((1, 1, C), cur),
            pl.BlockSpec((1, 1, C), prev),
            pl.BlockSpec((1, 1, C), cur),
            pl.BlockSpec((1, 1, C), prev),
        ],
        out_specs=pl.BlockSpec((1, C, W), cur),
        out_shape=jax.ShapeDtypeStruct((G, C, W), jnp.float32),
    )(sqv4, sqv4, sp3, sp3, sh3, sh3)

    N = R * B * H * T
    table = olse.reshape(N, W)
    undo_m = jnp.moveaxis(undo, 3, 1)
    bidx = jnp.arange(B, dtype=jnp.int32)
    hidx = jnp.arange(H, dtype=jnp.int32)
    ridx = jnp.arange(R, dtype=jnp.int32)
    base = ((ridx[:, None, None] * B + bidx[None, :, None]) * H
            + hidx[None, None, :]) * T
    gidx = (undo_m + base[:, None, :, :]).reshape(N)

    NW = 32
    per_w = N // NW
    ch = 512
    nch = per_w // ch
    gathered = _sc_unsort_gather(table, gidx, per_w, ch, nch)

    g3 = gathered.reshape(R, rows, H * W)

    out2d = pl.pallas_call(
        functools.partial(_combine_kernel, H=H, Dh=Dh),
        grid=(nblk,),
        in_specs=[
            pl.BlockSpec((R, BLK, H * W), lambda i: (0, i, 0)),
            pl.BlockSpec((E, E), lambda i: (0, 0)),
            pl.BlockSpec((1, E), lambda i: (0, 0)),
        ],
        out_specs=pl.BlockSpec((BLK, E), lambda i: (i, 0)),
        out_shape=jax.ShapeDtypeStruct((rows, E), jnp.float32),
    )(g3, Wout.T, bout.reshape(1, E))

    return out2d.reshape(T, B, E)
